# baseline (speedup 1.0000x reference)
import jax
import jax.numpy as jnp
from jax.experimental import pallas as pl

_B, _T = 16, 160000
_HOP = 320
_D = 1024
_CB_SIZE = 1024
_CB_DIM = 8
_F = _T // _HOP
_PAIR = 2

_HI = jax.lax.Precision.DEFAULT


def _dot(a, b):
    return jax.lax.dot_general(a, b, (((1,), (0,)), ((), ())),
                               precision=_HI, preferred_element_type=jnp.float32)


def _dot_t(a, b):
    return jax.lax.dot_general(a, b, (((1,), (1,)), ((), ())),
                               precision=_HI, preferred_element_type=jnp.float32)


def _chain(frames, W_enc, quants, iota):
    z = _dot(frames, W_enc)
    residual = z
    for stage, (Wi, cb, cbn, cbn_sq, Wo) in enumerate(quants):
        z_e = _dot(residual, Wi)
        enc = z_e / (jnp.sqrt(jnp.sum(z_e * z_e, -1, keepdims=True)) + 1e-8)
        dist = (jnp.sum(enc * enc, -1, keepdims=True)
                - 2.0 * _dot_t(enc, cbn)
                + cbn_sq)
        idx = jnp.argmin(dist, axis=-1)
        onehot = (iota == idx[:, None]).astype(jnp.float32)
        q = _dot(onehot, cb)
        out_i = _dot(q, Wo)
        if stage < 2:
            residual = residual - out_i
        else:
            return (z - residual) + out_i


def _rvq_kernel(wav_ref, W_enc_ref,
                Wi0_ref, cb0_ref, Wo0_ref,
                Wi1_ref, cb1_ref, Wo1_ref,
                Wi2_ref, cb2_ref, Wo2_ref,
                out_ref):
    W_enc = W_enc_ref[...]
    iota = jax.lax.broadcasted_iota(jnp.int32, (_F, _CB_SIZE), 1)
    quants = []
    for Wi_ref, cb_ref, Wo_ref in ((Wi0_ref, cb0_ref, Wo0_ref),
                                   (Wi1_ref, cb1_ref, Wo1_ref),
                                   (Wi2_ref, cb2_ref, Wo2_ref)):
        cb = cb_ref[...]
        cbn = cb / (jnp.sqrt(jnp.sum(cb * cb, -1, keepdims=True)) + 1e-8)
        cbn_sq = jnp.sum(cbn * cbn, -1)[None, :]
        quants.append((Wi_ref[...], cb, cbn, cbn_sq, Wo_ref[...]))
    for k in range(_PAIR):
        out_ref[0, k] = _chain(wav_ref[0, k], W_enc, quants, iota)


def kernel(wavs, W_enc, W_in_0, codebook_0, W_out_0,
           W_in_1, codebook_1, W_out_1, W_in_2, codebook_2, W_out_2):
    full = lambda shape: pl.BlockSpec(shape, lambda i: (0,) * len(shape))
    rows = wavs.reshape(_B // _PAIR, _PAIR, _F, _HOP)
    out = pl.pallas_call(
        _rvq_kernel,
        grid=(_B // _PAIR,),
        in_specs=[
            pl.BlockSpec((1, _PAIR, _F, _HOP), lambda i: (i, 0, 0, 0)),
            full((_HOP, _D)),
            full((_D, _CB_DIM)), full((_CB_SIZE, _CB_DIM)), full((_CB_DIM, _D)),
            full((_D, _CB_DIM)), full((_CB_SIZE, _CB_DIM)), full((_CB_DIM, _D)),
            full((_D, _CB_DIM)), full((_CB_SIZE, _CB_DIM)), full((_CB_DIM, _D)),
        ],
        out_specs=pl.BlockSpec((1, _PAIR, _F, _D), lambda i: (i, 0, 0, 0)),
        out_shape=jax.ShapeDtypeStruct((_B // _PAIR, _PAIR, _F, _D), jnp.float32),
    )(rows, W_enc,
      W_in_0, codebook_0, W_out_0,
      W_in_1, codebook_1, W_out_1,
      W_in_2, codebook_2, W_out_2)
    return out.reshape(_B, _F, _D)


# speedup vs baseline: 1.0797x; 1.0797x over previous
import jax
import jax.numpy as jnp
from jax.experimental import pallas as pl

_B, _T = 16, 160000
_HOP = 320
_D = 1024
_CB_SIZE = 1024
_CB_DIM = 8
_F = _T // _HOP
_PAIR = 2

_HI = jax.lax.Precision.DEFAULT


def _dot(a, b):
    return jax.lax.dot_general(a, b, (((1,), (0,)), ((), ())),
                               precision=_HI, preferred_element_type=jnp.float32)


def _dot_t(a, b):
    return jax.lax.dot_general(a, b, (((1,), (1,)), ((), ())),
                               precision=_HI, preferred_element_type=jnp.float32)


def _chain(frames, W_enc, quants, iota):
    z = _dot(frames, W_enc)
    residual = z
    for stage, (Wi, cb, cbn, cbn_sq, Wo) in enumerate(quants):
        z_e = _dot(residual, Wi)
        enc = z_e / (jnp.sqrt(jnp.sum(z_e * z_e, -1, keepdims=True)) + 1e-8)
        dist = (jnp.sum(enc * enc, -1, keepdims=True)
                - 2.0 * _dot_t(enc, cbn)
                + cbn_sq)
        idx = jnp.argmin(dist, axis=-1)
        onehot = (iota == idx[:, None]).astype(jnp.float32)
        q = _dot(onehot, cb)
        out_i = _dot(q, Wo)
        if stage < 2:
            residual = residual - out_i
        else:
            return (z - residual) + out_i


def _rvq_kernel(wav_ref, W_enc_ref,
                Wi0_ref, cb0_ref, Wo0_ref,
                Wi1_ref, cb1_ref, Wo1_ref,
                Wi2_ref, cb2_ref, Wo2_ref,
                out_ref):
    W_enc = W_enc_ref[...]
    iota = jax.lax.broadcasted_iota(jnp.int32, (_F, _CB_SIZE), 1)
    quants = []
    for Wi_ref, cb_ref, Wo_ref in ((Wi0_ref, cb0_ref, Wo0_ref),
                                   (Wi1_ref, cb1_ref, Wo1_ref),
                                   (Wi2_ref, cb2_ref, Wo2_ref)):
        cb = cb_ref[...]
        cbn = cb / (jnp.sqrt(jnp.sum(cb * cb, -1, keepdims=True)) + 1e-8)
        cbn_sq = jnp.sum(cbn * cbn, -1)[None, :]
        quants.append((Wi_ref[...], cb, cbn, cbn_sq, Wo_ref[...]))
    for k in range(_PAIR):
        out_ref[k] = _chain(wav_ref[k], W_enc, quants, iota)


def kernel(wavs, W_enc, W_in_0, codebook_0, W_out_0,
           W_in_1, codebook_1, W_out_1, W_in_2, codebook_2, W_out_2):
    full = lambda shape: pl.BlockSpec(shape, lambda i: (0,) * len(shape))
    rows = wavs.reshape(_B, _F, _HOP)
    out = pl.pallas_call(
        _rvq_kernel,
        grid=(_B // _PAIR,),
        in_specs=[
            pl.BlockSpec((_PAIR, _F, _HOP), lambda i: (i, 0, 0)),
            full((_HOP, _D)),
            full((_D, _CB_DIM)), full((_CB_SIZE, _CB_DIM)), full((_CB_DIM, _D)),
            full((_D, _CB_DIM)), full((_CB_SIZE, _CB_DIM)), full((_CB_DIM, _D)),
            full((_D, _CB_DIM)), full((_CB_SIZE, _CB_DIM)), full((_CB_DIM, _D)),
        ],
        out_specs=pl.BlockSpec((_PAIR, _F, _D), lambda i: (i, 0, 0)),
        out_shape=jax.ShapeDtypeStruct((_B, _F, _D), jnp.float32),
    )(rows, W_enc,
      W_in_0, codebook_0, W_out_0,
      W_in_1, codebook_1, W_out_1,
      W_in_2, codebook_2, W_out_2)
    return out
